# baseline (speedup 1.0000x reference)
import functools

import jax
import jax.numpy as jnp
from jax import lax
from jax.experimental import pallas as pl
from jax.experimental.pallas import tpu as pltpu
from jax.experimental.pallas import tpu_sc as plsc

CHUNK = 128
F32 = jnp.float32
_HIGH = lax.Precision.DEFAULT
NBLK = 2000
EBLK = 3200


def _sc_gather(table, catidx):
    E2 = catidx.shape[1]
    H = table.shape[1]
    mesh = plsc.VectorSubcoreMesh(core_axis_name="c", subcore_axis_name="s")

    @functools.partial(
        pl.kernel, out_type=jax.ShapeDtypeStruct((E2, H), F32), mesh=mesh)
    def k(t_hbm, i_hbm, g_hbm):
        def body(i_v, g_v):
            pltpu.sync_copy(t_hbm.at[i_v.at[0]], g_v)

        pltpu.emit_pipeline(
            body, grid=(E2 // CHUNK,),
            in_specs=[pl.BlockSpec((1, CHUNK), lambda i: (0, i))],
            out_specs=[pl.BlockSpec((CHUNK, H), lambda i: (i, 0))],
            core_axis_name=("c", "s"),
            dimension_semantics=(pltpu.PARALLEL,),
        )(i_hbm, g_hbm)

    return k(table, catidx)


def _tc_prep(x, Wl, bl, Wr, br):
    N, _ = x.shape
    H = Wl.shape[1]

    def body(x_ref, wl_ref, bl_ref, wr_ref, br_ref, t_ref):
        xv = x_ref[...]
        t_ref[:N, :] = jnp.dot(xv, wl_ref[...], precision=_HIGH,
                               preferred_element_type=F32) + bl_ref[...]
        t_ref[N:, :] = jnp.dot(xv, wr_ref[...], precision=_HIGH,
                               preferred_element_type=F32) + br_ref[...]

    return pl.pallas_call(
        body,
        out_shape=jax.ShapeDtypeStruct((2 * N, H), F32),
    )(x, Wl, bl.reshape(1, H), Wr, br.reshape(1, H))


def _tc_alpha_w(g, ea, We, att, with_extra):
    E2, H = g.shape
    E = E2 // 2
    DE = ea.shape[1]
    nblk = E // EBLK
    width = (H + 1 + DE + 1) if with_extra else (H + 1)

    def body(gs_ref, gd_ref, ea_ref, we_ref, att_ref, o_ref):
        gs = gs_ref[...]
        m = gs + gd_ref[...] + jnp.dot(
            ea_ref[...], we_ref[...], precision=_HIGH, preferred_element_type=F32)
        m = jnp.where(m >= 0, m, 0.2 * m)
        alpha = jnp.dot(m, att_ref[...], precision=_HIGH,
                        preferred_element_type=F32)
        ex = jnp.exp(alpha)
        if with_extra:
            o_ref[...] = jnp.concatenate(
                [ex * gs, ex, ea_ref[...],
                 jnp.ones((EBLK, 1), F32)], axis=1)
        else:
            o_ref[...] = jnp.concatenate([ex * gs, ex], axis=1)

    return pl.pallas_call(
        body,
        grid=(nblk,),
        in_specs=[pl.BlockSpec((EBLK, H), lambda i: (i, 0)),
                  pl.BlockSpec((EBLK, H), lambda i: (i + nblk, 0)),
                  pl.BlockSpec((EBLK, DE), lambda i: (i, 0)),
                  pl.BlockSpec((DE, H), lambda i: (0, 0)),
                  pl.BlockSpec((H, 1), lambda i: (0, 0))],
        out_specs=pl.BlockSpec((EBLK, width), lambda i: (i, 0)),
        out_shape=jax.ShapeDtypeStruct((E, width), F32),
    )(g, g, ea, We, att.reshape(H, 1))


def _selfloop_and_norm(num, d, la, xl, xr, we, att, bias):
    m = xl + xr + jnp.dot(la, we, precision=_HIGH, preferred_element_type=F32)
    m = jnp.where(m >= 0, m, 0.2 * m)
    a = jnp.dot(m, att, precision=_HIGH, preferred_element_type=F32)
    exs = jnp.exp(a)
    num = num + exs * xl
    d = d + exs
    return num / (d + 1e-16) + bias


def _tc_mid(acc, t1, We1, att1, bias1, n):
    H = t1.shape[1]
    nblk = n // NBLK
    DE = 16

    def body(acc_ref, xl_ref, xr_ref, we_ref, att_ref, b_ref,
             h_ref, la_ref, sums_ref):
        i = pl.program_id(0)
        a = acc_ref[...]
        deg = a[:, H + 1 + DE:H + 2 + DE]
        la = a[:, H + 1:H + 1 + DE] / jnp.clip(deg, 1.0, None)
        la_ref[...] = la
        h = _selfloop_and_norm(a[:, :H], a[:, H:H + 1], la, xl_ref[...],
                               xr_ref[...], we_ref[...], att_ref[...],
                               b_ref[...])
        h = jnp.where(h > 0, h, jnp.exp(h) - 1.0)
        h_ref[...] = h

        @pl.when(i == 0)
        def _():
            sums_ref[...] = jnp.zeros_like(sums_ref)

        sums_ref[0:1, :] += jnp.sum(h, axis=0, keepdims=True)
        sums_ref[1:2, :] += jnp.sum(h * h, axis=0, keepdims=True)

    width = H + 2 + DE
    return pl.pallas_call(
        body,
        grid=(nblk,),
        in_specs=[pl.BlockSpec((NBLK, width), lambda i: (i, 0)),
                  pl.BlockSpec((NBLK, H), lambda i: (i, 0)),
                  pl.BlockSpec((NBLK, H), lambda i: (i + nblk, 0)),
                  pl.BlockSpec((16, H), lambda i: (0, 0)),
                  pl.BlockSpec((H, 1), lambda i: (0, 0)),
                  pl.BlockSpec((1, H), lambda i: (0, 0))],
        out_specs=[pl.BlockSpec((NBLK, H), lambda i: (i, 0)),
                   pl.BlockSpec((NBLK, 16), lambda i: (i, 0)),
                   pl.BlockSpec((2, H), lambda i: (0, 0))],
        out_shape=(jax.ShapeDtypeStruct((n, H), F32),
                   jax.ShapeDtypeStruct((n, 16), F32),
                   jax.ShapeDtypeStruct((2, H), F32)),
    )(acc, t1, t1, We1, att1.reshape(H, 1), bias1.reshape(1, H))


def _tc_bnfold(sums, gamma, beta, Wl2, bl2, Wr2, br2, n):
    H = Wl2.shape[1]

    def body(s_ref, g_ref, be_ref, wl_ref, bl_ref, wr_ref, br_ref,
             wlf_ref, blf_ref, wrf_ref, brf_ref):
        mean = s_ref[0:1, :] / n
        var = s_ref[1:2, :] / n - mean * mean
        s = g_ref[...] * jax.lax.rsqrt(var + 1e-5)
        c = be_ref[...] - mean * s
        s_col = s.reshape(H, 1)
        wlf_ref[...] = wl_ref[...] * s_col
        blf_ref[...] = bl_ref[...] + jnp.dot(c, wl_ref[...], precision=_HIGH,
                                             preferred_element_type=F32)
        wrf_ref[...] = wr_ref[...] * s_col
        brf_ref[...] = br_ref[...] + jnp.dot(c, wr_ref[...], precision=_HIGH,
                                             preferred_element_type=F32)

    return pl.pallas_call(
        body,
        out_shape=(jax.ShapeDtypeStruct((H, H), F32),
                   jax.ShapeDtypeStruct((1, H), F32),
                   jax.ShapeDtypeStruct((H, H), F32),
                   jax.ShapeDtypeStruct((1, H), F32)),
    )(sums, gamma.reshape(1, H), beta.reshape(1, H), Wl2, bl2.reshape(1, H),
      Wr2, br2.reshape(1, H))


def _tc_proj2(h, Wlf, blf, Wrf, brf, n):
    H = h.shape[1]
    nblk = n // NBLK

    def body(h_ref, wl_ref, bl_ref, wr_ref, br_ref, xl_ref, xr_ref):
        hv = h_ref[...]
        xl_ref[...] = jnp.dot(hv, wl_ref[...], precision=_HIGH,
                              preferred_element_type=F32) + bl_ref[...]
        xr_ref[...] = jnp.dot(hv, wr_ref[...], precision=_HIGH,
                              preferred_element_type=F32) + br_ref[...]

    xl2, xr2 = pl.pallas_call(
        body,
        grid=(nblk,),
        in_specs=[pl.BlockSpec((NBLK, H), lambda i: (i, 0)),
                  pl.BlockSpec((H, H), lambda i: (0, 0)),
                  pl.BlockSpec((1, H), lambda i: (0, 0)),
                  pl.BlockSpec((H, H), lambda i: (0, 0)),
                  pl.BlockSpec((1, H), lambda i: (0, 0))],
        out_specs=[pl.BlockSpec((NBLK, H), lambda i: (i, 0)),
                   pl.BlockSpec((NBLK, H), lambda i: (i, 0))],
        out_shape=(jax.ShapeDtypeStruct((n, H), F32),
                   jax.ShapeDtypeStruct((n, H), F32)),
    )(h, Wlf, blf, Wrf, brf)
    return jnp.concatenate([xl2, xr2], axis=0)


def _tc_final(acc, la, t2, We2, att2, bias2, n):
    H = t2.shape[1]
    nblk = n // NBLK

    def body(acc_ref, la_ref, xl_ref, xr_ref, we_ref, att_ref, b_ref,
             out_ref):
        a = acc_ref[...]
        out_ref[...] = _selfloop_and_norm(
            a[:, :H], a[:, H:H + 1], la_ref[...], xl_ref[...], xr_ref[...],
            we_ref[...], att_ref[...], b_ref[...])

    return pl.pallas_call(
        body,
        grid=(nblk,),
        in_specs=[pl.BlockSpec((NBLK, H + 1), lambda i: (i, 0)),
                  pl.BlockSpec((NBLK, 16), lambda i: (i, 0)),
                  pl.BlockSpec((NBLK, H), lambda i: (i, 0)),
                  pl.BlockSpec((NBLK, H), lambda i: (i + nblk, 0)),
                  pl.BlockSpec((16, H), lambda i: (0, 0)),
                  pl.BlockSpec((H, 1), lambda i: (0, 0)),
                  pl.BlockSpec((1, H), lambda i: (0, 0))],
        out_specs=pl.BlockSpec((NBLK, H), lambda i: (i, 0)),
        out_shape=jax.ShapeDtypeStruct((n, H), F32),
    )(acc, la, t2, t2, We2, att2.reshape(H, 1), bias2.reshape(1, H))


def kernel(x, edge_index, edge_attr, Wl1, bl1, Wr1, br1, We1, att1, bias1,
           Wl2, bl2, Wr2, br2, We2, att2, bias2, gamma, beta):
    n = x.shape[0]
    E = edge_index.shape[1]
    H = Wl1.shape[1]

    src = edge_index[0].astype(jnp.int32)
    dst = edge_index[1].astype(jnp.int32)
    Eh = E // 4
    halves = []
    for lo in range(0, E, Eh):
        s_h = src[lo:lo + Eh].reshape(1, Eh)
        d_h = dst[lo:lo + Eh].reshape(1, Eh)
        halves.append((jnp.concatenate([s_h, d_h + n], axis=1),
                       dst[lo:lo + Eh], edge_attr[lo:lo + Eh]))

    def layer(t, We, att, width, with_extra):
        acc = jnp.zeros((n, width), F32)
        for catidx_h, dst_h, ea_h in halves:
            g = _sc_gather(t, catidx_h)
            p = _tc_alpha_w(g, ea_h, We, att, with_extra=with_extra)
            acc = acc.at[dst_h].add(p)
        return acc

    t1 = _tc_prep(x, Wl1, bl1, Wr1, br1)
    acc1 = layer(t1, We1, att1, H + 18, with_extra=True)
    h, la, sums = _tc_mid(acc1, t1, We1, att1, bias1, n)
    Wlf, blf, Wrf, brf = _tc_bnfold(sums, gamma, beta, Wl2, bl2, Wr2, br2, n)
    t2 = _tc_proj2(h, Wlf, blf, Wrf, brf, n)

    acc2 = layer(t2, We2, att2, H + 1, with_extra=False)
    return _tc_final(acc2, la, t2, We2, att2, bias2, n)


# speedup vs baseline: 1.0666x; 1.0666x over previous
import functools

import jax
import jax.numpy as jnp
from jax import lax
from jax.experimental import pallas as pl
from jax.experimental.pallas import tpu as pltpu
from jax.experimental.pallas import tpu_sc as plsc

CHUNK = 128
F32 = jnp.float32
_HIGH = lax.Precision.DEFAULT
NBLK = 2000
EBLK = 3200


def _sc_gather(table, catidx):
    E2 = catidx.shape[1]
    H = table.shape[1]
    mesh = plsc.VectorSubcoreMesh(core_axis_name="c", subcore_axis_name="s")

    @functools.partial(
        pl.kernel, out_type=jax.ShapeDtypeStruct((E2, H), F32), mesh=mesh)
    def k(t_hbm, i_hbm, g_hbm):
        def body(i_v, g_v):
            pltpu.sync_copy(t_hbm.at[i_v.at[0]], g_v)

        pltpu.emit_pipeline(
            body, grid=(E2 // CHUNK,),
            in_specs=[pl.BlockSpec((1, CHUNK), lambda i: (0, i))],
            out_specs=[pl.BlockSpec((CHUNK, H), lambda i: (i, 0))],
            core_axis_name=("c", "s"),
            dimension_semantics=(pltpu.PARALLEL,),
        )(i_hbm, g_hbm)

    return k(table, catidx)


def _tc_prep(x, Wl, bl, Wr, br):
    N, _ = x.shape
    H = Wl.shape[1]

    def body(x_ref, wl_ref, bl_ref, wr_ref, br_ref, t_ref):
        xv = x_ref[...]
        t_ref[:N, :] = jnp.dot(xv, wl_ref[...], precision=_HIGH,
                               preferred_element_type=F32) + bl_ref[...]
        t_ref[N:, :] = jnp.dot(xv, wr_ref[...], precision=_HIGH,
                               preferred_element_type=F32) + br_ref[...]

    return pl.pallas_call(
        body,
        out_shape=jax.ShapeDtypeStruct((2 * N, H), F32),
    )(x, Wl, bl.reshape(1, H), Wr, br.reshape(1, H))


def _tc_alpha_w(g, ea, We, att, with_extra):
    E2, H = g.shape
    E = E2 // 2
    DE = ea.shape[1]
    nblk = E // EBLK
    width = (H + 1 + DE + 1) if with_extra else (H + 1)

    def body(gs_ref, gd_ref, ea_ref, we_ref, att_ref, o_ref):
        gs = gs_ref[...]
        m = gs + gd_ref[...] + jnp.dot(
            ea_ref[...], we_ref[...], precision=_HIGH, preferred_element_type=F32)
        m = jnp.where(m >= 0, m, 0.2 * m)
        alpha = jnp.dot(m, att_ref[...], precision=_HIGH,
                        preferred_element_type=F32)
        ex = jnp.exp(alpha)
        if with_extra:
            o_ref[...] = jnp.concatenate(
                [ex * gs, ex, ea_ref[...],
                 jnp.ones((EBLK, 1), F32)], axis=1)
        else:
            o_ref[...] = jnp.concatenate([ex * gs, ex], axis=1)

    return pl.pallas_call(
        body,
        grid=(nblk,),
        in_specs=[pl.BlockSpec((EBLK, H), lambda i: (i, 0)),
                  pl.BlockSpec((EBLK, H), lambda i: (i + nblk, 0)),
                  pl.BlockSpec((EBLK, DE), lambda i: (i, 0)),
                  pl.BlockSpec((DE, H), lambda i: (0, 0)),
                  pl.BlockSpec((H, 1), lambda i: (0, 0))],
        out_specs=pl.BlockSpec((EBLK, width), lambda i: (i, 0)),
        out_shape=jax.ShapeDtypeStruct((E, width), F32),
    )(g, g, ea, We, att.reshape(H, 1))


def _selfloop_and_norm(num, d, la, xl, xr, we, att, bias):
    m = xl + xr + jnp.dot(la, we, precision=_HIGH, preferred_element_type=F32)
    m = jnp.where(m >= 0, m, 0.2 * m)
    a = jnp.dot(m, att, precision=_HIGH, preferred_element_type=F32)
    exs = jnp.exp(a)
    num = num + exs * xl
    d = d + exs
    return num / (d + 1e-16) + bias


def _tc_mid(acc, t1, We1, att1, bias1, n):
    H = t1.shape[1]
    nblk = n // NBLK
    DE = 16

    def body(acc_ref, xl_ref, xr_ref, we_ref, att_ref, b_ref,
             h_ref, la_ref, sums_ref):
        i = pl.program_id(0)
        a = acc_ref[...]
        deg = a[:, H + 1 + DE:H + 2 + DE]
        la = a[:, H + 1:H + 1 + DE] / jnp.clip(deg, 1.0, None)
        la_ref[...] = la
        h = _selfloop_and_norm(a[:, :H], a[:, H:H + 1], la, xl_ref[...],
                               xr_ref[...], we_ref[...], att_ref[...],
                               b_ref[...])
        h = jnp.where(h > 0, h, jnp.exp(h) - 1.0)
        h_ref[...] = h

        @pl.when(i == 0)
        def _():
            sums_ref[...] = jnp.zeros_like(sums_ref)

        sums_ref[0:1, :] += jnp.sum(h, axis=0, keepdims=True)
        sums_ref[1:2, :] += jnp.sum(h * h, axis=0, keepdims=True)

    width = H + 2 + DE
    return pl.pallas_call(
        body,
        grid=(nblk,),
        in_specs=[pl.BlockSpec((NBLK, width), lambda i: (i, 0)),
                  pl.BlockSpec((NBLK, H), lambda i: (i, 0)),
                  pl.BlockSpec((NBLK, H), lambda i: (i + nblk, 0)),
                  pl.BlockSpec((16, H), lambda i: (0, 0)),
                  pl.BlockSpec((H, 1), lambda i: (0, 0)),
                  pl.BlockSpec((1, H), lambda i: (0, 0))],
        out_specs=[pl.BlockSpec((NBLK, H), lambda i: (i, 0)),
                   pl.BlockSpec((NBLK, 16), lambda i: (i, 0)),
                   pl.BlockSpec((2, H), lambda i: (0, 0))],
        out_shape=(jax.ShapeDtypeStruct((n, H), F32),
                   jax.ShapeDtypeStruct((n, 16), F32),
                   jax.ShapeDtypeStruct((2, H), F32)),
    )(acc, t1, t1, We1, att1.reshape(H, 1), bias1.reshape(1, H))


def _tc_bnfold(sums, gamma, beta, Wl2, bl2, Wr2, br2, n):
    H = Wl2.shape[1]

    def body(s_ref, g_ref, be_ref, wl_ref, bl_ref, wr_ref, br_ref,
             wlf_ref, blf_ref, wrf_ref, brf_ref):
        mean = s_ref[0:1, :] / n
        var = s_ref[1:2, :] / n - mean * mean
        s = g_ref[...] * jax.lax.rsqrt(var + 1e-5)
        c = be_ref[...] - mean * s
        s_col = s.reshape(H, 1)
        wlf_ref[...] = wl_ref[...] * s_col
        blf_ref[...] = bl_ref[...] + jnp.dot(c, wl_ref[...], precision=_HIGH,
                                             preferred_element_type=F32)
        wrf_ref[...] = wr_ref[...] * s_col
        brf_ref[...] = br_ref[...] + jnp.dot(c, wr_ref[...], precision=_HIGH,
                                             preferred_element_type=F32)

    return pl.pallas_call(
        body,
        out_shape=(jax.ShapeDtypeStruct((H, H), F32),
                   jax.ShapeDtypeStruct((1, H), F32),
                   jax.ShapeDtypeStruct((H, H), F32),
                   jax.ShapeDtypeStruct((1, H), F32)),
    )(sums, gamma.reshape(1, H), beta.reshape(1, H), Wl2, bl2.reshape(1, H),
      Wr2, br2.reshape(1, H))


def _tc_proj2(h, Wlf, blf, Wrf, brf, n):
    H = h.shape[1]
    nblk = n // NBLK

    def body(h_ref, wl_ref, bl_ref, wr_ref, br_ref, xl_ref, xr_ref):
        hv = h_ref[...]
        xl_ref[...] = jnp.dot(hv, wl_ref[...], precision=_HIGH,
                              preferred_element_type=F32) + bl_ref[...]
        xr_ref[...] = jnp.dot(hv, wr_ref[...], precision=_HIGH,
                              preferred_element_type=F32) + br_ref[...]

    xl2, xr2 = pl.pallas_call(
        body,
        grid=(nblk,),
        in_specs=[pl.BlockSpec((NBLK, H), lambda i: (i, 0)),
                  pl.BlockSpec((H, H), lambda i: (0, 0)),
                  pl.BlockSpec((1, H), lambda i: (0, 0)),
                  pl.BlockSpec((H, H), lambda i: (0, 0)),
                  pl.BlockSpec((1, H), lambda i: (0, 0))],
        out_specs=[pl.BlockSpec((NBLK, H), lambda i: (i, 0)),
                   pl.BlockSpec((NBLK, H), lambda i: (i, 0))],
        out_shape=(jax.ShapeDtypeStruct((n, H), F32),
                   jax.ShapeDtypeStruct((n, H), F32)),
    )(h, Wlf, blf, Wrf, brf)
    return jnp.concatenate([xl2, xr2], axis=0)


def _tc_final(acc, la, t2, We2, att2, bias2, n):
    H = t2.shape[1]
    nblk = n // NBLK

    def body(acc_ref, la_ref, xl_ref, xr_ref, we_ref, att_ref, b_ref,
             out_ref):
        a = acc_ref[...]
        out_ref[...] = _selfloop_and_norm(
            a[:, :H], a[:, H:H + 1], la_ref[...], xl_ref[...], xr_ref[...],
            we_ref[...], att_ref[...], b_ref[...])

    return pl.pallas_call(
        body,
        grid=(nblk,),
        in_specs=[pl.BlockSpec((NBLK, H + 1), lambda i: (i, 0)),
                  pl.BlockSpec((NBLK, 16), lambda i: (i, 0)),
                  pl.BlockSpec((NBLK, H), lambda i: (i, 0)),
                  pl.BlockSpec((NBLK, H), lambda i: (i + nblk, 0)),
                  pl.BlockSpec((16, H), lambda i: (0, 0)),
                  pl.BlockSpec((H, 1), lambda i: (0, 0)),
                  pl.BlockSpec((1, H), lambda i: (0, 0))],
        out_specs=pl.BlockSpec((NBLK, H), lambda i: (i, 0)),
        out_shape=jax.ShapeDtypeStruct((n, H), F32),
    )(acc, la, t2, t2, We2, att2.reshape(H, 1), bias2.reshape(1, H))


def kernel(x, edge_index, edge_attr, Wl1, bl1, Wr1, br1, We1, att1, bias1,
           Wl2, bl2, Wr2, br2, We2, att2, bias2, gamma, beta):
    n = x.shape[0]
    E = edge_index.shape[1]
    H = Wl1.shape[1]

    src = edge_index[0].astype(jnp.int32)
    dst = edge_index[1].astype(jnp.int32)
    Eh = E // 2
    halves = []
    for lo in (0, Eh):
        s_h = src[lo:lo + Eh].reshape(1, Eh)
        d_h = dst[lo:lo + Eh].reshape(1, Eh)
        halves.append((jnp.concatenate([s_h, d_h + n], axis=1),
                       dst[lo:lo + Eh], edge_attr[lo:lo + Eh]))

    def layer(t, We, att, width, with_extra):
        acc = jnp.zeros((n, width), F32)
        for catidx_h, dst_h, ea_h in halves:
            g = _sc_gather(t, catidx_h)
            p = _tc_alpha_w(g, ea_h, We, att, with_extra=with_extra)
            acc = acc.at[dst_h].add(p)
        return acc

    t1 = _tc_prep(x, Wl1, bl1, Wr1, br1)
    acc1 = layer(t1, We1, att1, H + 18, with_extra=True)
    h, la, sums = _tc_mid(acc1, t1, We1, att1, bias1, n)
    Wlf, blf, Wrf, brf = _tc_bnfold(sums, gamma, beta, Wl2, bl2, Wr2, br2, n)
    t2 = _tc_proj2(h, Wlf, blf, Wrf, brf, n)

    acc2 = layer(t2, We2, att2, H + 1, with_extra=False)
    return _tc_final(acc2, la, t2, We2, att2, bias2, n)
